# baseline (speedup 1.0000x reference)
import functools

import jax
import jax.numpy as jnp
from jax import lax
from jax.experimental import pallas as pl
from jax.experimental.pallas import tpu as pltpu
from jax.experimental.pallas import tpu_sc as plsc
import numpy as np

TOPK = 8
_NP = 49
_DN = (((1,), (1,)), ((), ()))
_NW = 32
_L = 16


def _body(x_ref, noise_ref, W1_ref, b1_ref, W2_ref, b2_ref, W3_ref,
          b3_ref, W4_ref, b4_ref, Wu_ref, bu_ref, Wc_ref, bc_ref,
          sc_ref, h_ref, l2_ref, acc_ref):
    f32 = jnp.float32
    i = pl.program_id(0)

    @pl.when(i == 0)
    def _():
        acc_ref[...] = x_ref[0]
        l2 = jnp.float32(0.0)
        for r in (W1_ref, b1_ref, W2_ref, b2_ref, W3_ref, b3_ref, W4_ref,
                  b4_ref, Wu_ref, bu_ref, Wc_ref, bc_ref):
            v = r[...]
            l2 = l2 + jnp.sqrt(jnp.sum(v * v))
        l2_ref[...] = jnp.full((1, 1), 0.01, f32) * l2

    @pl.when(jnp.logical_and(i > 0, i < _NP))
    def _():
        acc_ref[...] = acc_ref[...] + x_ref[0]

    @pl.when(i == _NP)
    def _():
        def mm(x, w):
            return lax.dot_general(x, w, _DN, preferred_element_type=f32)

        pooled = acc_ref[...] * np.float32(1.0 / 49.0)
        h = jax.nn.relu(mm(pooled, W1_ref[...]) + b1_ref[...])
        h = jax.nn.relu(mm(h.astype(jnp.bfloat16), W2_ref[...]) + b2_ref[...])
        h = jax.nn.relu(mm(h, W3_ref[...]) + b3_ref[...])
        h = mm(h.astype(jnp.bfloat16), W4_ref[...]) + b4_ref[...]

        sc_ref[...] = mm(h, Wu_ref[...]) + bu_ref[...] + noise_ref[...]
        h_ref[...] = h


def _cls_body(h_ref, Wc_ref, bc_ref, cls_ref):
    cls_ref[...] = lax.dot_general(
        h_ref[...], Wc_ref[...], _DN,
        preferred_element_type=jnp.float32) + bc_ref[...]


def _route_body(scores_hbm, idx_hbm, rw_hbm, up_hbm, sc_v, idx_v, rw_v, up_v,
                *, rows):
    f32, i32 = jnp.float32, jnp.int32
    wid = lax.axis_index("s") * 2 + lax.axis_index("c")
    base = wid * rows
    pltpu.sync_copy(scores_hbm.at[pl.ds(base, rows)], sc_v)
    iota = lax.iota(i32, _L)
    neg = jnp.full((_L,), -jnp.inf, f32)

    gd = lax.GatherDimensionNumbers(offset_dims=(), collapsed_slice_dims=(0,),
                                    start_index_map=(0,))

    def allred(x, op):
        for stride in (8, 4, 2, 1):
            perm = lax.bitwise_xor(iota, stride).reshape(_L, 1)
            shuf = lax.gather(x, perm, gd, (1,),
                              mode=lax.GatherScatterMode.PROMISE_IN_BOUNDS)
            x = op(x, shuf)
        return x

    def row(r, us):
        v = [sc_v[r, pl.ds(j * _L, _L)] for j in range(4)]
        w = list(v)
        idxv = jnp.zeros((_L,), i32)
        m0v = jnp.zeros((_L,), f32)
        for k in range(TOPK):
            mv = allred(jnp.maximum(jnp.maximum(w[0], w[1]),
                                    jnp.maximum(w[2], w[3])), jnp.maximum)
            if k == 0:
                m0v = mv
            cand = [jnp.where(w[j] == mv, iota + (j * _L), 64)
                    for j in range(4)]
            cv = allred(jnp.minimum(jnp.minimum(cand[0], cand[1]),
                                    jnp.minimum(cand[2], cand[3])),
                        jnp.minimum)
            idxv = jnp.where(iota == k, cv, idxv)
            w = [jnp.where((iota + (j * _L)) == cv, neg, w[j])
                 for j in range(4)]
        e = [jnp.where(w[j] == neg, jnp.exp(v[j] - m0v), jnp.zeros((_L,), f32))
             for j in range(4)]
        sv = allred((e[0] + e[1]) + (e[2] + e[3]), jnp.add)
        idx_v[r] = idxv
        out_u = []
        for j in range(4):
            rwj = e[j] / sv
            rw_v[r, pl.ds(j * _L, _L)] = rwj
            out_u.append(us[j] + rwj)
        return tuple(out_u)

    z = jnp.zeros((_L,), f32)
    u = lax.fori_loop(0, rows, row, (z, z, z, z))
    for j in range(4):
        up_v[pl.ds(j * _L, _L)] = u[j]
    pltpu.sync_copy(idx_v, idx_hbm.at[pl.ds(base, rows)])
    pltpu.sync_copy(rw_v, rw_hbm.at[pl.ds(base, rows)])
    pltpu.sync_copy(up_v, up_hbm.at[wid])


def kernel(inputs, W1, b1, W2, b2, W3, b3, W4, b4, Wu, bu, Wc, bc):
    B, C, H, W = inputs.shape
    S = H * W
    E = Wu.shape[0]
    L = Wc.shape[0]
    f32 = jnp.float32

    noise = jax.random.normal(jax.random.key(1234), (B, E), f32) * 0.01
    x_pl = jnp.transpose(inputs, (2, 3, 0, 1)).reshape(S, B, C)

    def x_map(i):
        j = jnp.minimum(i, _NP - 1)
        return ((j % 7) * 7 + j // 7, 0, 0)

    full = lambda a: pl.BlockSpec(a.shape, lambda i: (0,) * a.ndim)
    b1r, b2r, b3r, b4r = (b.reshape(1, -1) for b in (b1, b2, b3, b4))
    bur, bcr = bu.reshape(1, -1), bc.reshape(1, -1)

    hid2 = W4.shape[0]
    scores, h, l2 = pl.pallas_call(
        _body,
        grid=(_NP + 1,),
        in_specs=[pl.BlockSpec((1, B, C), x_map), full(noise)] + [
            full(a) for a in (W1, b1r, W2, b2r, W3, b3r, W4, b4r,
                              Wu, bur, Wc, bcr)],
        out_specs=(full(jnp.zeros((B, E))), full(jnp.zeros((B, hid2))),
                   pl.BlockSpec((1, 1), lambda i: (0, 0))),
        out_shape=(jax.ShapeDtypeStruct((B, E), f32),
                   jax.ShapeDtypeStruct((B, hid2), f32),
                   jax.ShapeDtypeStruct((1, 1), f32)),
        scratch_shapes=[pltpu.VMEM((B, C), f32)],
    )(x_pl, noise, W1, b1r, W2, b2r, W3, b3r, W4, b4r, Wu, bur, Wc, bcr)

    cls = pl.pallas_call(
        _cls_body,
        grid=(1,),
        in_specs=[full(h), full(Wc), full(bcr)],
        out_specs=full(jnp.zeros((B, L))),
        out_shape=jax.ShapeDtypeStruct((B, L), f32),
    )(h, Wc, bcr)

    rows = B // _NW
    mesh = plsc.VectorSubcoreMesh(core_axis_name="c", subcore_axis_name="s")
    idx16, rw, up = pl.kernel(
        functools.partial(_route_body, rows=rows),
        mesh=mesh,
        out_type=[jax.ShapeDtypeStruct((B, _L), jnp.int32),
                  jax.ShapeDtypeStruct((B, E), f32),
                  jax.ShapeDtypeStruct((_NW, E), f32)],
        scratch_types=[pltpu.VMEM((rows, E), f32),
                       pltpu.VMEM((rows, _L), jnp.int32),
                       pltpu.VMEM((rows, E), f32),
                       pltpu.VMEM((E,), f32)],
    )(scores)

    usage = jnp.sum(up, axis=0) * np.float32(1.0 / B)
    return (rw, idx16[:, :TOPK], cls, l2.reshape(()), usage)


# speedup vs baseline: 1.0322x; 1.0322x over previous
import functools

import jax
import jax.numpy as jnp
from jax import lax
from jax.experimental import pallas as pl
from jax.experimental.pallas import tpu as pltpu
from jax.experimental.pallas import tpu_sc as plsc
import numpy as np

TOPK = 8
_NP = 49
_DN = (((1,), (1,)), ((), ()))
_NW = 32
_L = 16


def _body(x_ref, noise_ref, W1_ref, b1_ref, W2_ref, b2_ref, W3_ref,
          b3_ref, W4_ref, b4_ref, Wu_ref, bu_ref, Wc_ref, bc_ref,
          sc_ref, cls_ref, l2_ref, acc_ref):
    f32 = jnp.float32
    i = pl.program_id(0)

    @pl.when(i == 0)
    def _():
        acc_ref[...] = x_ref[0]
        l2 = jnp.float32(0.0)
        for r in (W1_ref, b1_ref, W2_ref, b2_ref, W3_ref, b3_ref, W4_ref,
                  b4_ref, Wu_ref, bu_ref, Wc_ref, bc_ref):
            v = r[...]
            l2 = l2 + jnp.sqrt(jnp.sum(v * v))
        l2_ref[...] = jnp.full((1, 1), 0.01, f32) * l2

    @pl.when(jnp.logical_and(i > 0, i < _NP))
    def _():
        acc_ref[...] = acc_ref[...] + x_ref[0]

    @pl.when(i == _NP)
    def _():
        def mm(x, w):
            return lax.dot_general(x, w, _DN, preferred_element_type=f32)

        pooled = acc_ref[...] * np.float32(1.0 / 49.0)
        h = jax.nn.relu(mm(pooled, W1_ref[...]) + b1_ref[...])
        h = jax.nn.relu(mm(h.astype(jnp.bfloat16), W2_ref[...]) + b2_ref[...])
        h = jax.nn.relu(mm(h, W3_ref[...]) + b3_ref[...])
        h = mm(h.astype(jnp.bfloat16), W4_ref[...]) + b4_ref[...]

        sc_ref[...] = mm(h, Wu_ref[...]) + bu_ref[...] + noise_ref[...]
        cls_ref[...] = mm(h, Wc_ref[...]) + bc_ref[...]


def _route_body(scores_hbm, idx_hbm, rw_hbm, up_hbm, sc_v, idx_v, rw_v, up_v,
                *, rows):
    f32, i32 = jnp.float32, jnp.int32
    wid = lax.axis_index("s") * 2 + lax.axis_index("c")
    base = wid * rows
    pltpu.sync_copy(scores_hbm.at[pl.ds(base, rows)], sc_v)
    iota = lax.iota(i32, _L)
    neg = jnp.full((_L,), -jnp.inf, f32)

    gd = lax.GatherDimensionNumbers(offset_dims=(), collapsed_slice_dims=(0,),
                                    start_index_map=(0,))

    def allred(x, op):
        for stride in (8, 4, 2, 1):
            perm = lax.bitwise_xor(iota, stride).reshape(_L, 1)
            shuf = lax.gather(x, perm, gd, (1,),
                              mode=lax.GatherScatterMode.PROMISE_IN_BOUNDS)
            x = op(x, shuf)
        return x

    def row(r, us):
        v = [sc_v[r, pl.ds(j * _L, _L)] for j in range(4)]
        w = list(v)
        idxv = jnp.zeros((_L,), i32)
        m0v = jnp.zeros((_L,), f32)
        for k in range(TOPK):
            mv = allred(jnp.maximum(jnp.maximum(w[0], w[1]),
                                    jnp.maximum(w[2], w[3])), jnp.maximum)
            if k == 0:
                m0v = mv
            cand = [jnp.where(w[j] == mv, iota + (j * _L), 64)
                    for j in range(4)]
            cv = allred(jnp.minimum(jnp.minimum(cand[0], cand[1]),
                                    jnp.minimum(cand[2], cand[3])),
                        jnp.minimum)
            idxv = jnp.where(iota == k, cv, idxv)
            w = [jnp.where((iota + (j * _L)) == cv, neg, w[j])
                 for j in range(4)]
        e = [jnp.where(w[j] == neg, jnp.exp(v[j] - m0v), jnp.zeros((_L,), f32))
             for j in range(4)]
        sv = allred((e[0] + e[1]) + (e[2] + e[3]), jnp.add)
        idx_v[r] = idxv
        out_u = []
        for j in range(4):
            rwj = e[j] / sv
            rw_v[r, pl.ds(j * _L, _L)] = rwj
            out_u.append(us[j] + rwj)
        return tuple(out_u)

    z = jnp.zeros((_L,), f32)
    u = lax.fori_loop(0, rows, row, (z, z, z, z))
    for j in range(4):
        up_v[pl.ds(j * _L, _L)] = u[j]
    pltpu.sync_copy(idx_v, idx_hbm.at[pl.ds(base, rows)])
    pltpu.sync_copy(rw_v, rw_hbm.at[pl.ds(base, rows)])
    pltpu.sync_copy(up_v, up_hbm.at[wid])


def kernel(inputs, W1, b1, W2, b2, W3, b3, W4, b4, Wu, bu, Wc, bc):
    B, C, H, W = inputs.shape
    S = H * W
    E = Wu.shape[0]
    L = Wc.shape[0]
    f32 = jnp.float32

    noise = jax.random.normal(jax.random.key(1234), (B, E), f32) * 0.01
    x_pl = jnp.transpose(inputs, (2, 3, 0, 1)).reshape(S, B, C)

    def x_map(i):
        j = jnp.minimum(i, _NP - 1)
        return ((j % 7) * 7 + j // 7, 0, 0)

    full = lambda a: pl.BlockSpec(a.shape, lambda i: (0,) * a.ndim)
    b1r, b2r, b3r, b4r = (b.reshape(1, -1) for b in (b1, b2, b3, b4))
    bur, bcr = bu.reshape(1, -1), bc.reshape(1, -1)

    scores, cls, l2 = pl.pallas_call(
        _body,
        grid=(_NP + 1,),
        in_specs=[pl.BlockSpec((1, B, C), x_map), full(noise)] + [
            full(a) for a in (W1, b1r, W2, b2r, W3, b3r, W4, b4r,
                              Wu, bur, Wc, bcr)],
        out_specs=(full(jnp.zeros((B, E))), full(jnp.zeros((B, L))),
                   pl.BlockSpec((1, 1), lambda i: (0, 0))),
        out_shape=(jax.ShapeDtypeStruct((B, E), f32),
                   jax.ShapeDtypeStruct((B, L), f32),
                   jax.ShapeDtypeStruct((1, 1), f32)),
        scratch_shapes=[pltpu.VMEM((B, C), f32)],
    )(x_pl, noise, W1, b1r, W2, b2r, W3, b3r, W4, b4r, Wu, bur, Wc, bcr)

    rows = B // _NW
    mesh = plsc.VectorSubcoreMesh(core_axis_name="c", subcore_axis_name="s")
    idx16, rw, up = pl.kernel(
        functools.partial(_route_body, rows=rows),
        mesh=mesh,
        out_type=[jax.ShapeDtypeStruct((B, _L), jnp.int32),
                  jax.ShapeDtypeStruct((B, E), f32),
                  jax.ShapeDtypeStruct((_NW, E), f32)],
        scratch_types=[pltpu.VMEM((rows, E), f32),
                       pltpu.VMEM((rows, _L), jnp.int32),
                       pltpu.VMEM((rows, E), f32),
                       pltpu.VMEM((E,), f32)],
    )(scores)

    usage = jnp.sum(up, axis=0) * np.float32(1.0 / B)
    return (rw, idx16[:, :TOPK], cls, l2.reshape(()), usage)


# speedup vs baseline: 1.1843x; 1.1474x over previous
import functools

import jax
import jax.numpy as jnp
from jax import lax
from jax.experimental import pallas as pl
from jax.experimental.pallas import tpu as pltpu
from jax.experimental.pallas import tpu_sc as plsc
import numpy as np

TOPK = 8
_NP = 49
_DN = (((1,), (1,)), ((), ()))
_NW = 32
_L = 16


def _body(x0_ref, x1_ref, x2_ref, x3_ref, x4_ref, x5_ref, x6_ref,
          noise_ref, W1_ref, b1_ref, W2_ref, b2_ref, W3_ref,
          b3_ref, W4_ref, b4_ref, Wu_ref, bu_ref, Wc_ref, bc_ref,
          sc_ref, cls_ref, l2_ref, acc_ref):
    f32 = jnp.float32
    i = pl.program_id(0)
    hf = pl.program_id(1)
    xs = (x0_ref, x1_ref, x2_ref, x3_ref, x4_ref, x5_ref, x6_ref)
    c2 = x0_ref.shape[2]
    csl = pl.ds(hf * c2, c2)

    @pl.when(i == 0)
    def _():
        a = xs[0][0]
        for r in xs[1:]:
            a = a + r[0]
        acc_ref[:, csl] = a

    @pl.when(jnp.logical_and(jnp.logical_and(i == 0, hf == 0),
                             jnp.bool_(True)))
    def _():
        l2 = jnp.float32(0.0)
        for r in (W1_ref, b1_ref, W2_ref, b2_ref, W3_ref, b3_ref, W4_ref,
                  b4_ref, Wu_ref, bu_ref, Wc_ref, bc_ref):
            v = r[...]
            l2 = l2 + jnp.sqrt(jnp.sum(v * v))
        l2_ref[...] = jnp.full((1, 1), 0.01, f32) * l2

    @pl.when(jnp.logical_and(i > 0, i < 7))
    def _():
        a = acc_ref[:, csl]
        for r in xs:
            a = a + r[0]
        acc_ref[:, csl] = a

    @pl.when(jnp.logical_and(i == 7, hf == 0))
    def _():
        def mm(x, w):
            return lax.dot_general(x, w, _DN, preferred_element_type=f32)

        pooled = acc_ref[...] * np.float32(1.0 / 49.0)
        h = jax.nn.relu(mm(pooled, W1_ref[...]) + b1_ref[...])
        h = jax.nn.relu(mm(h.astype(jnp.bfloat16), W2_ref[...]) + b2_ref[...])
        h = jax.nn.relu(mm(h, W3_ref[...]) + b3_ref[...])
        h = mm(h.astype(jnp.bfloat16), W4_ref[...]) + b4_ref[...]

        sc_ref[...] = mm(h, Wu_ref[...]) + bu_ref[...] + noise_ref[...]
        cls_ref[...] = mm(h, Wc_ref[...]) + bc_ref[...]


def _route_body(scores_hbm, idx_hbm, rw_hbm, up_hbm, sc_v, idx_v, rw_v, up_v,
                *, rows):
    f32, i32 = jnp.float32, jnp.int32
    wid = lax.axis_index("s") * 2 + lax.axis_index("c")
    base = wid * rows
    pltpu.sync_copy(scores_hbm.at[pl.ds(base, rows)], sc_v)
    iota = lax.iota(i32, _L)
    neg = jnp.full((_L,), -jnp.inf, f32)

    gd = lax.GatherDimensionNumbers(offset_dims=(), collapsed_slice_dims=(0,),
                                    start_index_map=(0,))

    def allred(x, op):
        for stride in (8, 4, 2, 1):
            perm = lax.bitwise_xor(iota, stride).reshape(_L, 1)
            shuf = lax.gather(x, perm, gd, (1,),
                              mode=lax.GatherScatterMode.PROMISE_IN_BOUNDS)
            x = op(x, shuf)
        return x

    def row(r, us):
        v = [sc_v[r, pl.ds(j * _L, _L)] for j in range(4)]
        w = list(v)
        idxv = jnp.zeros((_L,), i32)
        m0v = jnp.zeros((_L,), f32)
        for k in range(TOPK):
            mv = allred(jnp.maximum(jnp.maximum(w[0], w[1]),
                                    jnp.maximum(w[2], w[3])), jnp.maximum)
            if k == 0:
                m0v = mv
            cand = [jnp.where(w[j] == mv, iota + (j * _L), 64)
                    for j in range(4)]
            cv = allred(jnp.minimum(jnp.minimum(cand[0], cand[1]),
                                    jnp.minimum(cand[2], cand[3])),
                        jnp.minimum)
            idxv = jnp.where(iota == k, cv, idxv)
            w = [jnp.where((iota + (j * _L)) == cv, neg, w[j])
                 for j in range(4)]
        e = [jnp.where(w[j] == neg, jnp.exp(v[j] - m0v), jnp.zeros((_L,), f32))
             for j in range(4)]
        sv = allred((e[0] + e[1]) + (e[2] + e[3]), jnp.add)
        idx_v[r] = idxv
        out_u = []
        for j in range(4):
            rwj = e[j] / sv
            rw_v[r, pl.ds(j * _L, _L)] = rwj
            out_u.append(us[j] + rwj)
        return tuple(out_u)

    z = jnp.zeros((_L,), f32)
    u = lax.fori_loop(0, rows, row, (z, z, z, z))
    for j in range(4):
        up_v[pl.ds(j * _L, _L)] = u[j]
    pltpu.sync_copy(idx_v, idx_hbm.at[pl.ds(base, rows)])
    pltpu.sync_copy(rw_v, rw_hbm.at[pl.ds(base, rows)])
    pltpu.sync_copy(up_v, up_hbm.at[wid])


def kernel(inputs, W1, b1, W2, b2, W3, b3, W4, b4, Wu, bu, Wc, bc):
    B, C, H, W = inputs.shape
    S = H * W
    E = Wu.shape[0]
    L = Wc.shape[0]
    f32 = jnp.float32

    noise = jax.random.normal(jax.random.key(1234), (B, E), f32) * 0.01
    x_pl = jnp.transpose(inputs, (2, 3, 0, 1)).reshape(S, B, C)

    def x_map_for(h):
        return lambda i, f: (h * 7 + jnp.minimum(i, 6), 0,
                             jnp.where(i < 7, f, 1))

    full = lambda a: pl.BlockSpec(a.shape, lambda i, f: (0,) * a.ndim)
    b1r, b2r, b3r, b4r = (b.reshape(1, -1) for b in (b1, b2, b3, b4))
    bur, bcr = bu.reshape(1, -1), bc.reshape(1, -1)

    scores, cls, l2 = pl.pallas_call(
        _body,
        grid=(8, 2),
        in_specs=[pl.BlockSpec((1, B, C // 2), x_map_for(h))
                  for h in range(7)]
        + [full(noise)] + [
            full(a) for a in (W1, b1r, W2, b2r, W3, b3r, W4, b4r,
                              Wu, bur, Wc, bcr)],
        out_specs=(full(jnp.zeros((B, E))), full(jnp.zeros((B, L))),
                   pl.BlockSpec((1, 1), lambda i, f: (0, 0))),
        out_shape=(jax.ShapeDtypeStruct((B, E), f32),
                   jax.ShapeDtypeStruct((B, L), f32),
                   jax.ShapeDtypeStruct((1, 1), f32)),
        scratch_shapes=[pltpu.VMEM((B, C), f32)],
    )(x_pl, x_pl, x_pl, x_pl, x_pl, x_pl, x_pl,
      noise, W1, b1r, W2, b2r, W3, b3r, W4, b4r, Wu, bur, Wc, bcr)

    rows = B // _NW
    mesh = plsc.VectorSubcoreMesh(core_axis_name="c", subcore_axis_name="s")
    idx16, rw, up = pl.kernel(
        functools.partial(_route_body, rows=rows),
        mesh=mesh,
        out_type=[jax.ShapeDtypeStruct((B, _L), jnp.int32),
                  jax.ShapeDtypeStruct((B, E), f32),
                  jax.ShapeDtypeStruct((_NW, E), f32)],
        scratch_types=[pltpu.VMEM((rows, E), f32),
                       pltpu.VMEM((rows, _L), jnp.int32),
                       pltpu.VMEM((rows, E), f32),
                       pltpu.VMEM((E,), f32)],
    )(scores)

    usage = jnp.sum(up, axis=0) * np.float32(1.0 / B)
    return (rw, idx16[:, :TOPK], cls, l2.reshape(()), usage)
